# Initial kernel scaffold; baseline (speedup 1.0000x reference)
import jax
import jax.numpy as jnp
from jax import lax
from jax.experimental import pallas as pl
from jax.experimental.pallas import tpu as pltpu
from jax.experimental.pallas import tpu_sc as plsc

N = 10000
E = 320000
H = 128
L = 2
NB = 8
NC = 10

BE = 2560
NBLK = E // BE
SLAB = 512
NSLAB = E // SLAB
NW = 32
ZR = 125

_SC_MESH = plsc.VectorSubcoreMesh(core_axis_name="c", subcore_axis_name="s")


def _edge0_body(ea, bs, feW1, feb1, feW2, feb2, eW, eb, p1W, p1b, nemb, v0):
    u = jax.nn.gelu(bs[...] @ feW1[...] + feb1[...][None, :])
    b = jax.nn.gelu(u @ feW2[...] + feb2[...][None, :])
    M0 = eW[0] @ p1W[0]
    c0 = (eb[0][None, :] + nemb[...]) @ p1W[0] + p1b[0][None, :]
    a0 = ea[...] @ M0 + c0
    v0[...] = jnp.maximum(a0, 0.0) * b


def _edge1_body(g, ea, bs, feW1, feb1, feW2, feb2, eW, eb, p1W, p1b, v1):
    u = jax.nn.gelu(bs[...] @ feW1[...] + feb1[...][None, :])
    b = jax.nn.gelu(u @ feW2[...] + feb2[...][None, :])
    M1 = eW[1] @ p1W[1]
    c1 = eb[1][None, :] @ p1W[1] + p1b[1][None, :]
    a1 = ea[...] @ M1 + c1
    v1[...] = jnp.maximum(g[...] + a1, 0.0) * b


def _tail0_body(p, nemb, p2W, p2b, fW1, fb1, fW2, fb2, bng, bnb, p1W,
                h1o, hp1o, xtx0o):
    aggr = p[0] + p[1]
    t2 = jnp.maximum(nemb[...] @ p2W[0] + p2b[0][None, :], 0.0)
    y = aggr + t2
    mu = jnp.mean(y, axis=0, keepdims=True)
    yc = y - mu
    xtx0o[...] = lax.dot_general(yc, yc, (((0,), (0,)), ((), ()))) * (1.0 / N)
    z = jnp.maximum(y @ fW1[0] + fb1[0][None, :], 0.0)
    z = jnp.maximum(z @ fW2[0] + fb2[0][None, :], 0.0)
    muz = jnp.mean(z, axis=0, keepdims=True)
    zc = z - muz
    var = jnp.mean(zc * zc, axis=0, keepdims=True)
    h1 = zc * (bng[0][None, :] * lax.rsqrt(var + 1e-5)) + bnb[0][None, :]
    h1o[...] = h1
    hp1o[...] = h1 @ p1W[1]


def _tail1_body(p, h1, xtx0, p2W, p2b, fW1, fb1, fW2, fb2, bng, bnb,
                predW, predb, logitso, xtxo):
    aggr = p[0] + p[1]
    t2 = jnp.maximum(h1[...] @ p2W[1] + p2b[1][None, :], 0.0)
    y = aggr + t2
    mu = jnp.mean(y, axis=0, keepdims=True)
    yc = y - mu
    xtx1 = lax.dot_general(yc, yc, (((0,), (0,)), ((), ()))) * (1.0 / N)
    xtxo[...] = xtx0[...] + xtx1
    z = jnp.maximum(y @ fW1[1] + fb1[1][None, :], 0.0)
    z = jnp.maximum(z @ fW2[1] + fb2[1][None, :], 0.0)
    muz = jnp.mean(z, axis=0, keepdims=True)
    zc = z - muz
    var = jnp.mean(zc * zc, axis=0, keepdims=True)
    h2 = zc * (bng[1][None, :] * lax.rsqrt(var + 1e-5)) + bnb[1][None, :]
    s1 = jnp.sum(h1[...], axis=0, keepdims=True)
    s2 = jnp.sum(h2, axis=0, keepdims=True)
    logitso[...] = s1 @ predW[:H] + s2 @ predW[H:] + predb[...][None, :]


def _sc_scatter_body(v_hbm, dstr_hbm, zeros_hbm, out_hbm,
                     idx_v, rows_v, zbuf, accum):
    cid = lax.axis_index("c")
    sid = lax.axis_index("s")
    wid = sid * 2 + cid
    pltpu.sync_copy(zeros_hbm, zbuf)
    for t in range(5):
        pltpu.sync_copy(zbuf, accum.at[pl.ds(sid * 625 + t * ZR, ZR), :])
    plsc.subcore_barrier()
    nmine = jnp.where(wid < NSLAB - (NSLAB // NW) * NW, NSLAB // NW + 1,
                      NSLAB // NW)

    def step(t, carry):
        s = wid + t * NW
        pltpu.sync_copy(dstr_hbm.at[pl.ds(s * 4, 4), :], idx_v)
        pltpu.sync_copy(v_hbm.at[pl.ds(s * SLAB, SLAB), :], rows_v)
        for j in range(4):
            pltpu.sync_copy(rows_v.at[pl.ds(j * 128, 128), :],
                            accum.at[idx_v.at[j]], add=True)
        return carry

    lax.fori_loop(0, nmine, step, 0)
    plsc.subcore_barrier()
    pltpu.sync_copy(accum.at[pl.ds(sid * 625, 625), :],
                    out_hbm.at[cid, pl.ds(sid * 625, 625), :])


def _sc_gather_body(tab_hbm, srcr_hbm, g_hbm, idx_v, rows_v, sem):
    cid = lax.axis_index("c")
    sid = lax.axis_index("s")
    wid = sid * 2 + cid
    nmine = jnp.where(wid < NSLAB - (NSLAB // NW) * NW, NSLAB // NW + 1,
                      NSLAB // NW)

    def step(t, carry):
        s = wid + t * NW
        pltpu.sync_copy(srcr_hbm.at[pl.ds(s * 4, 4), :], idx_v)
        for j in range(4):
            pltpu.async_copy(tab_hbm.at[idx_v.at[j]],
                             rows_v.at[pl.ds(j * 128, 128), :], sem).wait()
        pltpu.sync_copy(rows_v, g_hbm.at[pl.ds(s * SLAB, SLAB), :])
        return carry

    lax.fori_loop(0, nmine, step, 0)


def _full(shape):
    return pl.BlockSpec(shape, lambda i: (0,) * len(shape))


def _edge_spec(cols):
    return pl.BlockSpec((BE, cols), lambda i: (i, 0))


def kernel(x, edge_index, edge_attr, bases, node_emb, fe_W1, fe_b1, fe_W2,
           fe_b2, edge_W, edge_b, pre1_W, pre1_b, pre2_W, pre2_b,
           ffn_W1, ffn_b1, ffn_W2, ffn_b2, bn_g, bn_b, pred_W, pred_b):
    f32 = jnp.float32
    src_r = edge_index[0].astype(jnp.int32).reshape(E // 128, 128)
    dst_r = edge_index[1].astype(jnp.int32).reshape(E // 128, 128)
    zeros_small = jnp.zeros((ZR, H), f32)

    wspecs = [
        _full((NB, H)), _full((H,)), _full((H, H)), _full((H,)),
        _full((L, 7, H)), _full((L, H)),
        _full((L, H, H)), _full((L, H)),
    ]

    v0 = pl.pallas_call(
        _edge0_body,
        grid=(NBLK,),
        in_specs=[_edge_spec(7), _edge_spec(NB)] + wspecs + [_full((1, H))],
        out_specs=_edge_spec(H),
        out_shape=jax.ShapeDtypeStruct((E, H), f32),
    )(edge_attr, bases, fe_W1, fe_b1, fe_W2, fe_b2, edge_W, edge_b,
      pre1_W, pre1_b, node_emb)

    scatter = pl.kernel(
        _sc_scatter_body,
        out_type=jax.ShapeDtypeStruct((2, N, H), f32),
        mesh=_SC_MESH,
        scratch_types=[
            pltpu.VMEM((4, 128), jnp.int32),
            pltpu.VMEM((SLAB, H), f32),
            pltpu.VMEM((ZR, H), f32),
            pltpu.VMEM_SHARED((N, H), f32),
        ],
    )

    p0 = scatter(v0, dst_r, zeros_small)

    h1, hp1, xtx0 = pl.pallas_call(
        _tail0_body,
        in_specs=[_full((2, N, H)), _full((1, H)),
                  _full((L, H, H)), _full((L, H)),
                  _full((L, H, H)), _full((L, H)),
                  _full((L, H, H)), _full((L, H)),
                  _full((L, H)), _full((L, H)),
                  _full((L, H, H))],
        out_specs=[_full((N, H)), _full((N, H)), _full((H, H))],
        out_shape=[jax.ShapeDtypeStruct((N, H), f32),
                   jax.ShapeDtypeStruct((N, H), f32),
                   jax.ShapeDtypeStruct((H, H), f32)],
    )(p0, node_emb, pre2_W, pre2_b, ffn_W1, ffn_b1, ffn_W2, ffn_b2,
      bn_g, bn_b, pre1_W)

    g = pl.kernel(
        _sc_gather_body,
        out_type=jax.ShapeDtypeStruct((E, H), f32),
        mesh=_SC_MESH,
        scratch_types=[
            pltpu.VMEM((4, 128), jnp.int32),
            pltpu.VMEM((SLAB, H), f32),
            pltpu.SemaphoreType.DMA,
        ],
    )(hp1, src_r)

    v1 = pl.pallas_call(
        _edge1_body,
        grid=(NBLK,),
        in_specs=[_edge_spec(H), _edge_spec(7), _edge_spec(NB)] + wspecs,
        out_specs=_edge_spec(H),
        out_shape=jax.ShapeDtypeStruct((E, H), f32),
    )(g, edge_attr, bases, fe_W1, fe_b1, fe_W2, fe_b2, edge_W, edge_b,
      pre1_W, pre1_b)

    p1 = scatter(v1, dst_r, zeros_small)

    logits, xtx = pl.pallas_call(
        _tail1_body,
        in_specs=[_full((2, N, H)), _full((N, H)), _full((H, H)),
                  _full((L, H, H)), _full((L, H)),
                  _full((L, H, H)), _full((L, H)),
                  _full((L, H, H)), _full((L, H)),
                  _full((L, H)), _full((L, H)),
                  _full((H * L, NC)), _full((NC,))],
        out_specs=[_full((1, NC)), _full((H, H))],
        out_shape=[jax.ShapeDtypeStruct((1, NC), f32),
                   jax.ShapeDtypeStruct((H, H), f32)],
    )(p1, h1, xtx0, pre2_W, pre2_b, ffn_W1, ffn_b1, ffn_W2, ffn_b2,
      bn_g, bn_b, pred_W, pred_b)

    return (logits, xtx)


# speedup vs baseline: 1.3597x; 1.3597x over previous
import jax
import jax.numpy as jnp
from jax import lax
from jax.experimental import pallas as pl
from jax.experimental.pallas import tpu as pltpu
from jax.experimental.pallas import tpu_sc as plsc

N = 10000
E = 320000
H = 128
L = 2
NB = 8
NC = 10

BE = 2560
NBLK = E // BE
SLAB = 256
NSLAB = E // SLAB
NW = 32


def _sc_mesh():
    return plsc.VectorSubcoreMesh(core_axis_name="c", subcore_axis_name="s")


def _edge0_body(ea, bs, feW1, feb1, feW2, feb2, eW, eb, p1W, p1b, nemb, v0):
    u = jax.nn.gelu(bs[...] @ feW1[...] + feb1[...][None, :])
    b = jax.nn.gelu(u @ feW2[...] + feb2[...][None, :])
    e = ea[...] @ eW[0] + eb[0][None, :]
    pos = nemb[...] + e
    t = pos @ p1W[0] + p1b[0][None, :]
    v0[...] = jnp.maximum(t, 0.0) * b


def _edge1_body(g, ea, bs, feW1, feb1, feW2, feb2, eW, eb, p1W, p1b, v1):
    u = jax.nn.gelu(bs[...] @ feW1[...] + feb1[...][None, :])
    b = jax.nn.gelu(u @ feW2[...] + feb2[...][None, :])
    e = ea[...] @ eW[1] + eb[1][None, :]
    pos = g[...] + e
    t = pos @ p1W[1] + p1b[1][None, :]
    v1[...] = jnp.maximum(t, 0.0) * b


def _cov_body(y, xtxo):
    mu = jnp.mean(y[...], axis=0, keepdims=True)
    yc = y[...] - mu
    xtxo[...] = lax.dot_general(yc, yc, (((0,), (0,)), ((), ()))) * (1.0 / N)


def _sc_gather_body(tab_hbm, srcr_hbm, g_hbm, idx_v, rows_v, sem):
    cid = lax.axis_index("c")
    sid = lax.axis_index("s")
    wid = sid * 2 + cid
    nmine = jnp.where(wid < NSLAB - (NSLAB // NW) * NW, NSLAB // NW + 1,
                      NSLAB // NW)

    def step(t, carry):
        s = wid + t * NW
        pltpu.sync_copy(srcr_hbm.at[s], idx_v)
        for j in range(SLAB // 128):
            pltpu.async_copy(tab_hbm.at[idx_v.at[j]],
                             rows_v.at[pl.ds(j * 128, 128), :], sem).wait()
        pltpu.sync_copy(rows_v, g_hbm.at[pl.ds(s * SLAB, SLAB), :])
        return carry

    lax.fori_loop(0, nmine, step, 0)


def _full(shape):
    return pl.BlockSpec(shape, lambda *_: (0,) * len(shape))


def _edge_spec(cols):
    return pl.BlockSpec((BE, cols), lambda i: (i, 0))


def kernel(x, edge_index, edge_attr, bases, node_emb, fe_W1, fe_b1, fe_W2,
           fe_b2, edge_W, edge_b, pre1_W, pre1_b, pre2_W, pre2_b,
           ffn_W1, ffn_b1, ffn_W2, ffn_b2, bn_g, bn_b, pred_W, pred_b):
    f32 = jnp.float32
    src = edge_index[0]
    dst = edge_index[1]
    src_r = src.astype(jnp.int32).reshape(NSLAB, SLAB // 128, 128)

    wspecs = [
        _full((NB, H)), _full((H,)), _full((H, H)), _full((H,)),
        _full((L, 7, H)), _full((L, H)),
        _full((L, H, H)), _full((L, H)),
    ]

    v0 = pl.pallas_call(
        _edge0_body,
        grid=(NBLK,),
        in_specs=[_edge_spec(7), _edge_spec(NB)] + wspecs + [_full((1, H))],
        out_specs=_edge_spec(H),
        out_shape=jax.ShapeDtypeStruct((E, H), f32),
    )(edge_attr, bases, fe_W1, fe_b1, fe_W2, fe_b2, edge_W, edge_b,
      pre1_W, pre1_b, node_emb)

    aggr0 = jnp.zeros((N, H), f32).at[dst].add(v0)

    h0 = node_emb[x]
    y0 = aggr0 + jax.nn.relu(h0 @ pre2_W[0] + pre2_b[0])
    xtx0 = pl.pallas_call(
        _cov_body,
        in_specs=[_full((N, H))],
        out_specs=_full((H, H)),
        out_shape=jax.ShapeDtypeStruct((H, H), f32),
    )(y0)
    z0 = jax.nn.relu(y0 @ ffn_W1[0] + ffn_b1[0])
    z0 = jax.nn.relu(z0 @ ffn_W2[0] + ffn_b2[0])
    mu0 = jnp.mean(z0, axis=0)
    var0 = jnp.var(z0, axis=0)
    h1 = (z0 - mu0) / jnp.sqrt(var0 + 1e-5) * bn_g[0] + bn_b[0]

    g = pl.kernel(
        _sc_gather_body,
        out_type=jax.ShapeDtypeStruct((E, H), f32),
        mesh=_sc_mesh(),
        scratch_types=[
            pltpu.VMEM((SLAB // 128, 128), jnp.int32),
            pltpu.VMEM((SLAB, H), f32),
            pltpu.SemaphoreType.DMA,
        ],
    )(h1, src_r)

    v1 = pl.pallas_call(
        _edge1_body,
        grid=(NBLK,),
        in_specs=[_edge_spec(H), _edge_spec(7), _edge_spec(NB)] + wspecs,
        out_specs=_edge_spec(H),
        out_shape=jax.ShapeDtypeStruct((E, H), f32),
    )(g, edge_attr, bases, fe_W1, fe_b1, fe_W2, fe_b2, edge_W, edge_b,
      pre1_W, pre1_b)

    aggr1 = jnp.zeros((N, H), f32).at[dst].add(v1)

    y1 = aggr1 + jax.nn.relu(h1 @ pre2_W[1] + pre2_b[1])
    xtx1 = pl.pallas_call(
        _cov_body,
        in_specs=[_full((N, H))],
        out_specs=_full((H, H)),
        out_shape=jax.ShapeDtypeStruct((H, H), f32),
    )(y1)
    xtx = xtx0 + xtx1
    z1 = jax.nn.relu(y1 @ ffn_W1[1] + ffn_b1[1])
    z1 = jax.nn.relu(z1 @ ffn_W2[1] + ffn_b2[1])
    mu1 = jnp.mean(z1, axis=0)
    var1 = jnp.var(z1, axis=0)
    h2 = (z1 - mu1) / jnp.sqrt(var1 + 1e-5) * bn_g[1] + bn_b[1]

    xcat = jnp.concatenate([h1, h2], axis=1)
    hg = jnp.sum(xcat, axis=0, keepdims=True)
    logits = hg @ pred_W + pred_b
    return (logits, xtx)


# speedup vs baseline: 1.3908x; 1.0229x over previous
import jax
import jax.numpy as jnp
from jax import lax
from jax.experimental import pallas as pl
from jax.experimental.pallas import tpu as pltpu
from jax.experimental.pallas import tpu_sc as plsc

N = 10000
E = 320000
H = 128
L = 2
NB = 8
NC = 10

BE = 2560
NBLK = E // BE
SLAB = 512
NSLAB = E // SLAB
NW = 32


def _sc_mesh():
    return plsc.VectorSubcoreMesh(core_axis_name="c", subcore_axis_name="s")


def _edge0_body(ea, bs, feW1, feb1, feW2, feb2, eW, eb, p1W, p1b, nemb, v0):
    u = jax.nn.gelu(bs[...] @ feW1[...] + feb1[...][None, :])
    b = jax.nn.gelu(u @ feW2[...] + feb2[...][None, :])
    e = ea[...] @ eW[0] + eb[0][None, :]
    pos = nemb[...] + e
    t = pos @ p1W[0] + p1b[0][None, :]
    v0[...] = jnp.maximum(t, 0.0) * b


def _edge1_body(g, ea, bs, feW1, feb1, feW2, feb2, eW, eb, p1W, p1b, v1):
    u = jax.nn.gelu(bs[...] @ feW1[...] + feb1[...][None, :])
    b = jax.nn.gelu(u @ feW2[...] + feb2[...][None, :])
    e = ea[...] @ eW[1] + eb[1][None, :]
    pos = g[...] + e
    t = pos @ p1W[1] + p1b[1][None, :]
    v1[...] = jnp.maximum(t, 0.0) * b


def _cov_body(y, xtxo):
    mu = jnp.mean(y[...], axis=0, keepdims=True)
    yc = y[...] - mu
    xtxo[...] = lax.dot_general(yc, yc, (((0,), (0,)), ((), ()))) * (1.0 / N)


def _sc_gather_body(tab_hbm, srcr_hbm, g_hbm, idx_v, rows_v, sem):
    cid = lax.axis_index("c")
    sid = lax.axis_index("s")
    wid = sid * 2 + cid
    nmine = jnp.where(wid < NSLAB - (NSLAB // NW) * NW, NSLAB // NW + 1,
                      NSLAB // NW)

    def step(t, carry):
        s = wid + t * NW
        pltpu.sync_copy(srcr_hbm.at[s], idx_v)
        copies = [
            pltpu.async_copy(tab_hbm.at[idx_v.at[j]],
                             rows_v.at[pl.ds(j * 128, 128), :], sem)
            for j in range(SLAB // 128)
        ]
        for c in copies:
            c.wait()
        pltpu.sync_copy(rows_v, g_hbm.at[pl.ds(s * SLAB, SLAB), :])
        return carry

    lax.fori_loop(0, nmine, step, 0)


def _full(shape):
    return pl.BlockSpec(shape, lambda *_: (0,) * len(shape))


def _edge_spec(cols):
    return pl.BlockSpec((BE, cols), lambda i: (i, 0))


def kernel(x, edge_index, edge_attr, bases, node_emb, fe_W1, fe_b1, fe_W2,
           fe_b2, edge_W, edge_b, pre1_W, pre1_b, pre2_W, pre2_b,
           ffn_W1, ffn_b1, ffn_W2, ffn_b2, bn_g, bn_b, pred_W, pred_b):
    f32 = jnp.float32
    src = edge_index[0]
    dst = edge_index[1]
    src_r = src.astype(jnp.int32).reshape(NSLAB, SLAB // 128, 128)

    wspecs = [
        _full((NB, H)), _full((H,)), _full((H, H)), _full((H,)),
        _full((L, 7, H)), _full((L, H)),
        _full((L, H, H)), _full((L, H)),
    ]

    v0 = pl.pallas_call(
        _edge0_body,
        grid=(NBLK,),
        in_specs=[_edge_spec(7), _edge_spec(NB)] + wspecs + [_full((1, H))],
        out_specs=_edge_spec(H),
        out_shape=jax.ShapeDtypeStruct((E, H), f32),
    )(edge_attr, bases, fe_W1, fe_b1, fe_W2, fe_b2, edge_W, edge_b,
      pre1_W, pre1_b, node_emb)

    aggr0 = jnp.zeros((N, H), f32).at[dst].add(v0)

    h0 = node_emb[x]
    y0 = aggr0 + jax.nn.relu(h0 @ pre2_W[0] + pre2_b[0])
    xtx0 = pl.pallas_call(
        _cov_body,
        in_specs=[_full((N, H))],
        out_specs=_full((H, H)),
        out_shape=jax.ShapeDtypeStruct((H, H), f32),
    )(y0)
    z0 = jax.nn.relu(y0 @ ffn_W1[0] + ffn_b1[0])
    z0 = jax.nn.relu(z0 @ ffn_W2[0] + ffn_b2[0])
    mu0 = jnp.mean(z0, axis=0)
    var0 = jnp.var(z0, axis=0)
    h1 = (z0 - mu0) / jnp.sqrt(var0 + 1e-5) * bn_g[0] + bn_b[0]

    g = pl.kernel(
        _sc_gather_body,
        out_type=jax.ShapeDtypeStruct((E, H), f32),
        mesh=_sc_mesh(),
        scratch_types=[
            pltpu.VMEM((SLAB // 128, 128), jnp.int32),
            pltpu.VMEM((SLAB, H), f32),
            pltpu.SemaphoreType.DMA,
        ],
    )(h1, src_r)

    v1 = pl.pallas_call(
        _edge1_body,
        grid=(NBLK,),
        in_specs=[_edge_spec(H), _edge_spec(7), _edge_spec(NB)] + wspecs,
        out_specs=_edge_spec(H),
        out_shape=jax.ShapeDtypeStruct((E, H), f32),
    )(g, edge_attr, bases, fe_W1, fe_b1, fe_W2, fe_b2, edge_W, edge_b,
      pre1_W, pre1_b)

    aggr1 = jnp.zeros((N, H), f32).at[dst].add(v1)

    y1 = aggr1 + jax.nn.relu(h1 @ pre2_W[1] + pre2_b[1])
    xtx1 = pl.pallas_call(
        _cov_body,
        in_specs=[_full((N, H))],
        out_specs=_full((H, H)),
        out_shape=jax.ShapeDtypeStruct((H, H), f32),
    )(y1)
    xtx = xtx0 + xtx1
    z1 = jax.nn.relu(y1 @ ffn_W1[1] + ffn_b1[1])
    z1 = jax.nn.relu(z1 @ ffn_W2[1] + ffn_b2[1])
    mu1 = jnp.mean(z1, axis=0)
    var1 = jnp.var(z1, axis=0)
    h2 = (z1 - mu1) / jnp.sqrt(var1 + 1e-5) * bn_g[1] + bn_b[1]

    xcat = jnp.concatenate([h1, h2], axis=1)
    hg = jnp.sum(xcat, axis=0, keepdims=True)
    logits = hg @ pred_W + pred_b
    return (logits, xtx)
